# Initial kernel scaffold; baseline (speedup 1.0000x reference)
import jax
import jax.numpy as jnp
from jax.experimental import pallas as pl


def kernel(ego_fut_preds, lane_preds, lane_score_preds, weight):
    raise NotImplementedError("write your pallas kernel here")


# speedup vs baseline: 1.8833x; 1.8833x over previous
import functools

import jax
import jax.numpy as jnp
from jax import lax
from jax.experimental import pallas as pl
from jax.experimental.pallas import tpu as pltpu
from jax.experimental.pallas import tpu_sc as plsc

_B, _T, _V, _P = 1024, 6, 100, 20
_VP = _V * _P
_X0, _Y0, _X1, _Y1 = -15.0, -30.0, 15.0, 30.0
_MAP_THRESH = 0.5
_DIS_THRESH = 1.0
_LOSS_WEIGHT = 1.0
_PENALTY = 1.0e12

_BB = 64
_BC = 256

def _tri(strict):
    r = lax.broadcasted_iota(jnp.int32, (_T, _T), 0)
    c = lax.broadcasted_iota(jnp.int32, (_T, _T), 1)
    return jnp.where(r < c if strict else r <= c, 1.0, 0.0).astype(jnp.float32)

_NC, _NS, _L = 2, 16, 16
_NW = _NC * _NS
_RPW = (_B * _T) // _NW
_CHUNKS = 2
_SUB = _B // _NW // _CHUNKS
_CH = _RPW // _CHUNKS
_G = _CH // _L


def _nn_body(lx_ref, ly_ref, srep_ref, egox_ref, egoy_ref, d2_ref, idx_ref):
    tx = lx_ref[...] * (_X1 - _X0) + _X0
    ty = ly_ref[...] * (_Y1 - _Y0) + _Y0
    pen = jnp.where(srep_ref[...] < _MAP_THRESH, _PENALTY, 0.0)
    egox = egox_ref[...]
    egoy = egoy_ref[...]
    tri = _tri(False)
    px = jnp.dot(egox, tri, preferred_element_type=jnp.float32)
    py = jnp.dot(egoy, tri, preferred_element_type=jnp.float32)
    jota = lax.broadcasted_iota(jnp.int32, (_BB, _VP), 1)
    row = (pl.program_id(0) * _BB
           + lax.broadcasted_iota(jnp.int32, (_BB, 1), 0))
    for t in range(_T):
        dx = tx - px[:, t:t + 1]
        dy = ty - py[:, t:t + 1]
        d2 = dx * dx + dy * dy + pen
        m = jnp.min(d2, axis=1, keepdims=True)
        d2_ref[:, t:t + 1] = m
        j = jnp.min(jnp.where(d2 <= m, jota, _VP), axis=1, keepdims=True)
        idx_ref[:, t:t + 1] = (row % _SUB) * _VP + (j // _P) * _P


def _geom_body(bdx_ref, bdy_ref, d2_ref, egox_ref, egoy_ref, w_ref, acc_ref):
    egox = egox_ref[...]
    egoy = egoy_ref[...]
    tri, tris = _tri(False), _tri(True)
    px = jnp.dot(egox, tri, preferred_element_type=jnp.float32)
    py = jnp.dot(egoy, tri, preferred_element_type=jnp.float32)
    esx = jnp.dot(egox, tris, preferred_element_type=jnp.float32)
    esy = jnp.dot(egoy, tris, preferred_element_type=jnp.float32)
    inters = []
    for t in range(_T):
        bx = bdx_ref[:, t * _P:(t + 1) * _P] * (_X1 - _X0) + _X0
        by = bdy_ref[:, t * _P:(t + 1) * _P] * (_Y1 - _Y0) + _Y0
        sxx, exx = bx[:, :_P - 1], bx[:, 1:]
        syy, eyy = by[:, :_P - 1], by[:, 1:]
        dx1 = px[:, t:t + 1] - esx[:, t:t + 1]
        dy1 = py[:, t:t + 1] - esy[:, t:t + 1]
        dx2 = exx - sxx
        dy2 = eyy - syy
        det = dx1 * dy2 - dx2 * dy1
        par = det == 0.0
        dets = jnp.where(par, 1.0, det)
        rx = sxx - esx[:, t:t + 1]
        ry = syy - esy[:, t:t + 1]
        t1 = (rx * dy2 - ry * dx2) / dets
        t2 = (rx * dy1 - ry * dx1) / dets
        ok = ((t1 >= 0.0) & (t1 <= 1.0) & (t2 >= 0.0) & (t2 <= 1.0)
              & jnp.logical_not(par))
        inters.append(jnp.any(ok, axis=1, keepdims=True).astype(jnp.int32))
    inter = jnp.concatenate(inters, axis=1)
    tio = lax.broadcasted_iota(jnp.int32, (_BC, _T), 1)
    ft = jnp.min(jnp.where(inter > 0, tio, _T), axis=1, keepdims=True)
    md = jnp.sqrt(d2_ref[...])
    loss = jnp.where(md > _DIS_THRESH, 0.0, _DIS_THRESH - md)
    loss = jnp.where(tio >= ft, 0.0, loss)
    s = jnp.sum(loss * w_ref[...])

    @pl.when(pl.program_id(0) == 0)
    def _():
        acc_ref[0, 0] = 0.0

    acc_ref[0, 0] += s


def _nn_search(lane_x, lane_y, scores_rep, ego_x, ego_y):
    grid = _B // _BB
    return pl.pallas_call(
        _nn_body,
        grid=(grid,),
        in_specs=[
            pl.BlockSpec((_BB, _VP), lambda i: (i, 0)),
            pl.BlockSpec((_BB, _VP), lambda i: (i, 0)),
            pl.BlockSpec((_BB, _VP), lambda i: (i, 0)),
            pl.BlockSpec((_BB, _T), lambda i: (i, 0)),
            pl.BlockSpec((_BB, _T), lambda i: (i, 0)),
        ],
        out_specs=[
            pl.BlockSpec((_BB, _T), lambda i: (i, 0)),
            pl.BlockSpec((_BB, _T), lambda i: (i, 0)),
        ],
        out_shape=[
            jax.ShapeDtypeStruct((_B, _T), jnp.float32),
            jax.ShapeDtypeStruct((_B, _T), jnp.int32),
        ],
    )(lane_x, lane_y, scores_rep, ego_x, ego_y)


def _sc_gather(lane_x_flat, lane_y_flat, off):
    mesh = plsc.VectorSubcoreMesh(core_axis_name="c", subcore_axis_name="s")
    sub_elems = _SUB * _VP

    @functools.partial(
        pl.kernel,
        mesh=mesh,
        out_type=[
            jax.ShapeDtypeStruct((_B * _T, _P), jnp.float32),
            jax.ShapeDtypeStruct((_B * _T, _P), jnp.float32),
        ],
        scratch_types=[
            pltpu.VMEM((_RPW,), jnp.int32),
            pltpu.VMEM((sub_elems,), jnp.float32),
            pltpu.VMEM((sub_elems,), jnp.float32),
            pltpu.VMEM((_CH, _P), jnp.float32),
            pltpu.VMEM((_CH, _P), jnp.float32),
            pltpu.SemaphoreType.DMA,
        ],
        compiler_params=pltpu.CompilerParams(needs_layout_passes=False),
    )
    def k(xtab, ytab, off_hbm, outx, outy, offv, xch, ych, ox, oy, sem):
        wid = lax.axis_index("s") * _NC + lax.axis_index("c")
        rbase = wid * _RPW
        pltpu.sync_copy(off_hbm.at[pl.ds(rbase, _RPW)], offv)
        lane = lax.iota(jnp.int32, _L)
        for c in range(_CHUNKS):
            ebase = (wid * _CHUNKS + c) * sub_elems
            cpx = pltpu.async_copy(xtab.at[pl.ds(ebase, sub_elems)], xch, sem)
            cpy = pltpu.async_copy(ytab.at[pl.ds(ebase, sub_elems)], ych, sem)
            cpx.wait()
            cpy.wait()
            for g in range(_G):
                base = offv[pl.ds(c * _CH + g * _L, _L)]
                qrow = g * _L + lane
                for p in range(_P):
                    xv = plsc.load_gather(xch, [base + p])
                    yv = plsc.load_gather(ych, [base + p])
                    pv = jnp.full((_L,), p, jnp.int32)
                    plsc.store_scatter(ox, [qrow, pv], xv)
                    plsc.store_scatter(oy, [qrow, pv], yv)
            pltpu.sync_copy(ox, outx.at[pl.ds(rbase + c * _CH, _CH)])
            pltpu.sync_copy(oy, outy.at[pl.ds(rbase + c * _CH, _CH)])

    return k(lane_x_flat, lane_y_flat, off)


def _geom_loss(bdx, bdy, d2min, ego_x, ego_y, weight):
    grid = _B // _BC
    return pl.pallas_call(
        _geom_body,
        grid=(grid,),
        in_specs=[
            pl.BlockSpec((_BC, _T * _P), lambda i: (i, 0)),
            pl.BlockSpec((_BC, _T * _P), lambda i: (i, 0)),
            pl.BlockSpec((_BC, _T), lambda i: (i, 0)),
            pl.BlockSpec((_BC, _T), lambda i: (i, 0)),
            pl.BlockSpec((_BC, _T), lambda i: (i, 0)),
            pl.BlockSpec((_BC, _T), lambda i: (i, 0)),
        ],
        out_specs=pl.BlockSpec(memory_space=pltpu.SMEM),
        out_shape=jax.ShapeDtypeStruct((1, 1), jnp.float32),
    )(bdx, bdy, d2min, ego_x, ego_y, weight)


def kernel(ego_fut_preds, lane_preds, lane_score_preds, weight):
    lane_x = lane_preds[..., 0].reshape(_B, _VP)
    lane_y = lane_preds[..., 1].reshape(_B, _VP)
    scores_rep = jnp.repeat(lane_score_preds[..., 2], _P, axis=1)
    ego_x = ego_fut_preds[..., 0]
    ego_y = ego_fut_preds[..., 1]

    d2min, idx = _nn_search(lane_x, lane_y, scores_rep, ego_x, ego_y)
    bdx, bdy = _sc_gather(lane_x.reshape(_B * _VP),
                          lane_y.reshape(_B * _VP),
                          idx.reshape(_B * _T))
    acc = _geom_loss(bdx.reshape(_B, _T * _P), bdy.reshape(_B, _T * _P),
                     d2min, ego_x, ego_y, weight)
    return _LOSS_WEIGHT * acc[0, 0] / (_B * _T)
